# baseline (speedup 1.0000x reference)
import functools

import jax
import jax.numpy as jnp
from jax import lax
from jax.experimental import pallas as pl
from jax.experimental.pallas import tpu as pltpu
from jax.experimental.pallas import tpu_sc as plsc

_D = 256
_V = 8192
_NQ = 8192
_BETA = 0.25
_QBLK = 1024

_NC = 2
_NS = 16
_NW = _NC * _NS
_BPW = _NQ // _NW
_CH = 64
_NCH = _BPW // _CH


def _vq_dist_kernel(z_ref, w_ref, wsq_ref, idx_ref, dsum_ref):
    i = pl.program_id(0)
    z = z_ref[...]
    w = w_ref[...]
    zsq = jnp.sum(z * z, axis=1, keepdims=True)
    zw2 = lax.dot_general(z + z, w, (((1,), (1,)), ((), ())),
                          preferred_element_type=jnp.float32)
    dist = (zsq + wsq_ref[...]) - zw2
    m = jnp.min(dist, axis=1, keepdims=True)
    ids = lax.broadcasted_iota(jnp.int32, (1, dist.shape[1]), 1).astype(jnp.float32)
    idx_f = jnp.min(jnp.where(dist == m, ids, jnp.float32(3e38)), axis=1)
    idx_ref[...] = idx_f.astype(jnp.int32)

    @pl.when(i == 0)
    def _init():
        dsum_ref[0, 0] = 0.0

    dsum_ref[0, 0] += jnp.sum(m)


def _dist_argmin(z_flat, W, wsq):
    grid = (_NQ // _QBLK,)
    return pl.pallas_call(
        _vq_dist_kernel,
        grid=grid,
        in_specs=[
            pl.BlockSpec((_QBLK, _D), lambda i: (i, 0)),
            pl.BlockSpec((_V, _D), lambda i: (0, 0)),
            pl.BlockSpec((1, _V), lambda i: (0, 0)),
        ],
        out_specs=[
            pl.BlockSpec((_QBLK,), lambda i: (i,)),
            pl.BlockSpec((1, 1), lambda i: (0, 0), memory_space=pltpu.SMEM),
        ],
        out_shape=[
            jax.ShapeDtypeStruct((_NQ,), jnp.int32),
            jax.ShapeDtypeStruct((1, 1), jnp.float32),
        ],
    )(z_flat, W, wsq)


@functools.cache
def _make_sc_gather():
    mesh = plsc.VectorSubcoreMesh(core_axis_name="c", subcore_axis_name="s")

    @functools.partial(
        pl.kernel,
        mesh=mesh,
        out_type=jax.ShapeDtypeStruct((_NQ, _D), jnp.float32),
        scratch_types=[
            pltpu.VMEM((_NCH, _CH), jnp.int32),
            pltpu.VMEM((_BPW, _D), jnp.float32),
            pltpu.SemaphoreType.DMA,
            pltpu.SemaphoreType.DMA,
        ],
    )
    def gather(w_hbm, idx_hbm, out_hbm, idx_v, rows_v, gsem, ssem):
        wid = lax.axis_index("s") * _NC + lax.axis_index("c")
        base = wid * _BPW
        pltpu.sync_copy(idx_hbm.at[wid], idx_v)
        gathers = [
            pltpu.async_copy(w_hbm.at[idx_v.at[c]],
                             rows_v.at[pl.ds(c * _CH, _CH)], gsem)
            for c in range(_NCH)
        ]
        scatters = []
        for c in range(_NCH):
            gathers[c].wait()
            scatters.append(
                pltpu.async_copy(rows_v.at[pl.ds(c * _CH, _CH)],
                                 out_hbm.at[pl.ds(base + c * _CH, _CH)], ssem))
        for cp in scatters:
            cp.wait()

    return gather


def kernel(inputs, W):
    z = inputs
    B, ih, iw, D = z.shape
    z_flat = jnp.reshape(z, (-1, D))
    wsq = jnp.sum(W ** 2, axis=-1)[None, :]
    idx_flat, dsum = _dist_argmin(z_flat, W, wsq)
    min_ind = jnp.reshape(idx_flat, (B, ih * iw))
    zq = _make_sc_gather()(W, jnp.reshape(idx_flat, (_NW, _NCH, _CH)))
    zq_st = jnp.reshape(zq, (B, ih, iw, D))
    loss = dsum[0, 0] * ((1.0 + _BETA) / (B * ih * iw * D))
    return (zq_st, min_ind, loss)


# speedup vs baseline: 1.0167x; 1.0167x over previous
import functools

import jax
import jax.numpy as jnp
from jax import lax
from jax.experimental import pallas as pl
from jax.experimental.pallas import tpu as pltpu
from jax.experimental.pallas import tpu_sc as plsc

_D = 256
_V = 8192
_NQ = 8192
_BETA = 0.25
_QBLK = 1024

_NC = 2
_NS = 16
_NW = _NC * _NS
_BPW = _NQ // _NW
_CH = 128
_NCH = _BPW // _CH


def _vq_dist_kernel(z_ref, w_ref, wsq_ref, idx_ref, dsum_ref):
    i = pl.program_id(0)
    z = z_ref[...]
    w = w_ref[...]
    zsq = jnp.sum(z * z, axis=1, keepdims=True)
    zw2 = lax.dot_general(z + z, w, (((1,), (1,)), ((), ())),
                          preferred_element_type=jnp.float32)
    dist = (zsq + wsq_ref[...]) - zw2
    m = jnp.min(dist, axis=1, keepdims=True)
    ids = lax.broadcasted_iota(jnp.int32, (1, dist.shape[1]), 1).astype(jnp.float32)
    idx_f = jnp.min(jnp.where(dist == m, ids, jnp.float32(3e38)), axis=1)
    idx_ref[...] = idx_f.astype(jnp.int32)

    @pl.when(i == 0)
    def _init():
        dsum_ref[0, 0] = 0.0

    dsum_ref[0, 0] += jnp.sum(m)


def _dist_argmin(z_flat, W, wsq):
    grid = (_NQ // _QBLK,)
    return pl.pallas_call(
        _vq_dist_kernel,
        grid=grid,
        in_specs=[
            pl.BlockSpec((_QBLK, _D), lambda i: (i, 0)),
            pl.BlockSpec((_V, _D), lambda i: (0, 0)),
            pl.BlockSpec((1, _V), lambda i: (0, 0)),
        ],
        out_specs=[
            pl.BlockSpec((_QBLK,), lambda i: (i,)),
            pl.BlockSpec((1, 1), lambda i: (0, 0), memory_space=pltpu.SMEM),
        ],
        out_shape=[
            jax.ShapeDtypeStruct((_NQ,), jnp.int32),
            jax.ShapeDtypeStruct((1, 1), jnp.float32),
        ],
    )(z_flat, W, wsq)


@functools.cache
def _make_sc_gather():
    mesh = plsc.VectorSubcoreMesh(core_axis_name="c", subcore_axis_name="s")

    @functools.partial(
        pl.kernel,
        mesh=mesh,
        out_type=jax.ShapeDtypeStruct((_NQ, _D), jnp.float32),
        scratch_types=[
            pltpu.VMEM((_NCH, _CH), jnp.int32),
            pltpu.VMEM((_BPW, _D), jnp.float32),
            pltpu.SemaphoreType.DMA,
            pltpu.SemaphoreType.DMA,
        ],
    )
    def gather(w_hbm, idx_hbm, out_hbm, idx_v, rows_v, gsem, ssem):
        wid = lax.axis_index("s") * _NC + lax.axis_index("c")
        base = wid * _BPW
        pltpu.sync_copy(idx_hbm.at[wid], idx_v)
        gathers = [
            pltpu.async_copy(w_hbm.at[idx_v.at[c]],
                             rows_v.at[pl.ds(c * _CH, _CH)], gsem)
            for c in range(_NCH)
        ]
        scatters = []
        for c in range(_NCH):
            gathers[c].wait()
            scatters.append(
                pltpu.async_copy(rows_v.at[pl.ds(c * _CH, _CH)],
                                 out_hbm.at[pl.ds(base + c * _CH, _CH)], ssem))
        for cp in scatters:
            cp.wait()

    return gather


def kernel(inputs, W):
    z = inputs
    B, ih, iw, D = z.shape
    z_flat = jnp.reshape(z, (-1, D))
    wsq = jnp.sum(W ** 2, axis=-1)[None, :]
    idx_flat, dsum = _dist_argmin(z_flat, W, wsq)
    min_ind = jnp.reshape(idx_flat, (B, ih * iw))
    zq = _make_sc_gather()(W, jnp.reshape(idx_flat, (_NW, _NCH, _CH)))
    zq_st = jnp.reshape(zq, (B, ih, iw, D))
    loss = dsum[0, 0] * ((1.0 + _BETA) / (B * ih * iw * D))
    return (zq_st, min_ind, loss)
